# baseline (speedup 1.0000x reference)
import functools

import jax
import jax.numpy as jnp
from jax import lax
from jax.experimental import pallas as pl
from jax.experimental.pallas import tpu as pltpu
from jax.experimental.pallas import tpu_sc as plsc

VOCAB = 100000
EMBED_DIM = 64
BATCH = 4096
HIST = 50

NTOT = BATCH * HIST
NUM_WORKERS = 32
PER_WORKER = NTOT // NUM_WORKERS
RUN = 256
NUNITS = PER_WORKER // RUN
NLANE = 16

_MESH = plsc.VectorSubcoreMesh(core_axis_name="c", subcore_axis_name="s")


@functools.partial(
    pl.kernel,
    mesh=_MESH,
    out_type=jax.ShapeDtypeStruct((HIST, EMBED_DIM, BATCH), jnp.float32),
    scratch_types=[
        pltpu.VMEM((NUNITS, RUN), jnp.int32),
        pltpu.VMEM((2, RUN, EMBED_DIM), jnp.float32),
        pltpu.VMEM((2, EMBED_DIM, RUN), jnp.float32),
        pltpu.SemaphoreType.DMA,
        pltpu.SemaphoreType.DMA,
        pltpu.SemaphoreType.DMA,
        pltpu.SemaphoreType.DMA,
    ],
    compiler_params=pltpu.CompilerParams(use_tc_tiling_on_sc=False, needs_layout_passes=False),
)
def _embed_gather(idx_hbm, table_hbm, out_hbm, idx_v, rows_v, tbuf_v,
                  gsem0, gsem1, ssem0, ssem1):
    wid = lax.axis_index("s") * 2 + lax.axis_index("c")
    f_base = wid * PER_WORKER
    gsems = (gsem0, gsem1)
    ssems = (ssem0, ssem1)
    pltpu.sync_copy(idx_hbm.at[wid], idx_v)
    lanes = lax.iota(jnp.int32, NLANE)

    def start_gather(k):
        bb = k % 2
        return pltpu.async_copy(
            table_hbm.at[idx_v.at[k]], rows_v.at[bb], gsems[bb])

    def transpose_unit(k):
        bb = k % 2
        rows = rows_v.at[bb]
        tbuf = tbuf_v.at[bb]

        def col_body(c, carry):
            cvec = jnp.full((NLANE,), c, jnp.int32)
            for g in range(RUN // NLANE):
                vec = plsc.load_gather(rows, [g * NLANE + lanes, cvec])
                tbuf[c, pl.ds(g * NLANE, NLANE)] = vec
            return carry

        lax.fori_loop(0, EMBED_DIM, col_body, 0)

    def start_store(k):
        bb = k % 2
        f0 = f_base + k * RUN
        h = f0 // BATCH
        b0 = f0 % BATCH
        return pltpu.async_copy(
            tbuf_v.at[bb], out_hbm.at[h, :, pl.ds(b0, RUN)], ssems[bb])

    gathers = [None] * NUNITS
    stores = [None] * NUNITS
    gathers[0] = start_gather(0)
    for k in range(NUNITS):
        if k + 1 < NUNITS:
            gathers[k + 1] = start_gather(k + 1)
        gathers[k].wait()
        if k - 2 >= 0:
            stores[k - 2].wait()
        transpose_unit(k)
        stores[k] = start_store(k)
    stores[NUNITS - 2].wait()
    stores[NUNITS - 1].wait()


def kernel(input_tensor, table):
    idx = input_tensor.T.astype(jnp.int32).reshape(NUM_WORKERS, NUNITS, RUN)
    out_phys = _embed_gather(idx, table)
    return jnp.transpose(out_phys, (2, 0, 1))
